# baseline (speedup 1.0000x reference)
import functools

import jax
import jax.numpy as jnp
from jax import lax
from jax.experimental import pallas as pl
from jax.experimental.pallas import tpu as pltpu
from jax.experimental.pallas import tpu_sc as plsc

N_NODES = 10000
D = 128
TIMESTEP = 3

ROW_BLK = 2000

NTILES = 16
CHUNK = 128
NROWS = 160000 // CHUNK
RPT = 80
EROWS = NTILES * RPT
NPAD = 10240
SLICE = NPAD // NTILES


def _tc_body(x_ref, deg_ref, win_ref, wout_ref, wih_ref, whh_ref,
             bih_ref, bhh_ref, out_ref):
    xb = x_ref[...]
    dn = (((1,), (1,)), ((), ()))
    pin = lax.dot_general(xb, win_ref[...], dn,
                          preferred_element_type=jnp.float32)
    pout = lax.dot_general(xb, wout_ref[...], dn,
                           preferred_element_type=jnp.float32)
    a_in = jnp.where(deg_ref[:, 0:1] > 0.0, pin, 0.0)
    a_out = jnp.where(deg_ref[:, 1:2] > 0.0, pout, 0.0)
    gf = jnp.maximum(a_in, a_out)
    gi = lax.dot_general(gf, wih_ref[...], dn,
                         preferred_element_type=jnp.float32) + bih_ref[...]
    h = xb
    for _ in range(TIMESTEP):
        gh = lax.dot_general(h, whh_ref[...], dn,
                             preferred_element_type=jnp.float32) + bhh_ref[...]
        r = jax.nn.sigmoid(gi[:, :D] + gh[:, :D])
        z = jax.nn.sigmoid(gi[:, D:2 * D] + gh[:, D:2 * D])
        n = jnp.tanh(gi[:, 2 * D:] + r * gh[:, 2 * D:])
        h = (1.0 - z) * n + z * h
    out_ref[...] = h


def _dense_stage(x, degT, We_in, We_out, w_ih, w_hh, bih2, bhh2):
    n = x.shape[0]
    grid = n // ROW_BLK
    wspec = lambda a: pl.BlockSpec(a.shape, lambda i: (0, 0))
    return pl.pallas_call(
        _tc_body,
        grid=(grid,),
        in_specs=[
            pl.BlockSpec((ROW_BLK, D), lambda i: (i, 0)),
            pl.BlockSpec((ROW_BLK, 2), lambda i: (i, 0)),
            wspec(We_in), wspec(We_out), wspec(w_ih), wspec(w_hh),
            wspec(bih2), wspec(bhh2),
        ],
        out_specs=pl.BlockSpec((ROW_BLK, D), lambda i: (i, 0)),
        out_shape=jax.ShapeDtypeStruct((n, D), jnp.float32),
    )(x, degT, We_in, We_out, w_ih, w_hh, bih2, bhh2)


def _sc_mask_body(dst_ref, out_ref, idx_v, ones_v, zeros_v, shared,
                  idx_sem, sc_sem):
    c = lax.axis_index("c")
    s = lax.axis_index("s")
    off = s * RPT
    idx_cp = pltpu.make_async_copy(dst_ref.at[c, pl.ds(off, RPT)], idx_v,
                                   idx_sem)
    idx_cp.start()
    for i in range(CHUNK // 16):
        ones_v[pl.ds(i * 16, 16)] = jnp.full((16,), 1.0, jnp.float32)
    for i in range(SLICE // 16):
        zeros_v[pl.ds(i * 16, 16)] = jnp.zeros((16,), jnp.float32)
    pltpu.sync_copy(zeros_v, shared.at[pl.ds(s * SLICE, SLICE)])
    plsc.subcore_barrier()
    idx_cp.wait()
    cps = [pltpu.async_copy(ones_v, shared.at[idx_v.at[j]], sc_sem, add=True)
           for j in range(RPT)]
    for cp in cps:
        cp.wait()
    plsc.subcore_barrier()
    pltpu.sync_copy(shared.at[pl.ds(s * SLICE, SLICE)],
                    out_ref.at[c, pl.ds(s * SLICE, SLICE)])


_sc_masks = pl.kernel(
    _sc_mask_body,
    out_type=jax.ShapeDtypeStruct((2, NPAD), jnp.float32),
    mesh=plsc.VectorSubcoreMesh(core_axis_name="c", subcore_axis_name="s"),
    scratch_types=[
        pltpu.VMEM((RPT, CHUNK), jnp.int32),
        pltpu.VMEM((CHUNK,), jnp.float32),
        pltpu.VMEM((SLICE,), jnp.float32),
        pltpu.VMEM_SHARED((NPAD,), jnp.float32),
        pltpu.SemaphoreType.DMA,
        pltpu.SemaphoreType.DMA,
    ],
)


def kernel(x, We_in, We_out, w_ih, w_hh, b_ih, b_hh, edge_index_in,
           edge_index_out):
    n = x.shape[0]
    pad = jnp.full((EROWS - NROWS, CHUNK), N_NODES, jnp.int32)
    dst3 = jnp.stack(
        [jnp.concatenate([edge_index_in[1].reshape(NROWS, CHUNK), pad]),
         jnp.concatenate([edge_index_out[1].reshape(NROWS, CHUNK), pad])]
    ).astype(jnp.int32)
    deg = _sc_masks(dst3)
    degT = deg.T[:n]
    return _dense_stage(x, degT, We_in, We_out, w_ih, w_hh,
                        b_ih.reshape(1, 3 * D), b_hh.reshape(1, 3 * D))


# speedup vs baseline: 1.2668x; 1.2668x over previous
import functools

import jax
import jax.numpy as jnp
from jax import lax
from jax.experimental import pallas as pl
from jax.experimental.pallas import tpu as pltpu
from jax.experimental.pallas import tpu_sc as plsc

N_NODES = 10000
D = 128
TIMESTEP = 3

ROW_BLK = 2000

NTILES = 16
CHUNK = 128
EDGES = 160000
EPT = EDGES // NTILES
NCHUNK = EPT // CHUNK
TAIL = EPT - NCHUNK * CHUNK
NBUF = 4
NPAD = 10240
SLICE = NPAD // NTILES


def _tc_body(x_ref, deg_ref, win_ref, wout_ref, wih_ref, whh_ref,
             bih_ref, bhh_ref, out_ref, degt_scr):
    i = pl.program_id(0)
    xb = x_ref[...]
    dn = (((1,), (1,)), ((), ()))

    @pl.when(i == 0)
    def _():
        degt_scr[...] = jnp.transpose(deg_ref[...])

    degb = degt_scr[pl.ds(i * ROW_BLK, ROW_BLK), :]
    pin = lax.dot_general(xb, win_ref[...], dn,
                          preferred_element_type=jnp.float32)
    pout = lax.dot_general(xb, wout_ref[...], dn,
                           preferred_element_type=jnp.float32)
    a_in = jnp.where(degb[:, 0:1] > 0.0, pin, 0.0)
    a_out = jnp.where(degb[:, 1:2] > 0.0, pout, 0.0)
    gf = jnp.maximum(a_in, a_out)
    gi = lax.dot_general(gf, wih_ref[...], dn,
                         preferred_element_type=jnp.float32) + bih_ref[...]
    h = xb
    for _ in range(TIMESTEP):
        gh = lax.dot_general(h, whh_ref[...], dn,
                             preferred_element_type=jnp.float32) + bhh_ref[...]
        r = jax.nn.sigmoid(gi[:, :D] + gh[:, :D])
        z = jax.nn.sigmoid(gi[:, D:2 * D] + gh[:, D:2 * D])
        n = jnp.tanh(gi[:, 2 * D:] + r * gh[:, 2 * D:])
        h = (1.0 - z) * n + z * h
    out_ref[...] = h


def _dense_stage(x, deg, We_in, We_out, w_ih, w_hh, b_ih, b_hh):
    n = x.shape[0]
    grid = n // ROW_BLK
    wspec = lambda a: pl.BlockSpec(a.shape, lambda i: (0,) * a.ndim)
    return pl.pallas_call(
        _tc_body,
        grid=(grid,),
        in_specs=[
            pl.BlockSpec((ROW_BLK, D), lambda i: (i, 0)),
            wspec(deg),
            wspec(We_in), wspec(We_out), wspec(w_ih), wspec(w_hh),
            wspec(b_ih), wspec(b_hh),
        ],
        out_specs=pl.BlockSpec((ROW_BLK, D), lambda i: (i, 0)),
        out_shape=jax.ShapeDtypeStruct((n, D), jnp.float32),
        scratch_shapes=[pltpu.VMEM((NPAD, 2), jnp.float32)],
    )(x, deg, We_in, We_out, w_ih, w_hh, b_ih, b_hh)


def _sc_mask_body(ein_ref, eout_ref, out_ref, idx_v, ones_v, zeros_v, shared,
                  idx_sem, sc_sem):
    c = lax.axis_index("c")
    s = lax.axis_index("s")
    base = s * EPT

    idx_cp_in = pltpu.make_async_copy(ein_ref.at[pl.ds(base, EPT)],
                                      idx_v, idx_sem)
    idx_cp_out = pltpu.make_async_copy(eout_ref.at[pl.ds(base, EPT)],
                                       idx_v, idx_sem)

    @pl.when(c == 0)
    def _():
        idx_cp_in.start()

    @pl.when(c == 1)
    def _():
        idx_cp_out.start()

    for i in range(CHUNK // 16):
        ones_v[pl.ds(i * 16, 16)] = jnp.full((16,), 1.0, jnp.float32)
    for i in range(SLICE // 16):
        zeros_v[pl.ds(i * 16, 16)] = jnp.zeros((16,), jnp.float32)
    pltpu.sync_copy(zeros_v, shared.at[pl.ds(s * SLICE, SLICE)])
    plsc.subcore_barrier()
    idx_cp_in.wait()

    cps = [pltpu.async_copy(ones_v,
                            shared.at[idx_v.at[pl.ds(j * CHUNK, CHUNK)]],
                            sc_sem, add=True)
           for j in range(NCHUNK)]
    cps.append(pltpu.async_copy(
        ones_v.at[pl.ds(0, TAIL)],
        shared.at[idx_v.at[pl.ds(NCHUNK * CHUNK, TAIL)]],
        sc_sem, add=True))
    for cp in cps:
        cp.wait()

    plsc.subcore_barrier()
    pltpu.sync_copy(shared.at[pl.ds(s * SLICE, SLICE)],
                    out_ref.at[c, pl.ds(s * SLICE, SLICE)])


_sc_masks = pl.kernel(
    _sc_mask_body,
    out_type=jax.ShapeDtypeStruct((2, NPAD), jnp.float32),
    mesh=plsc.VectorSubcoreMesh(core_axis_name="c", subcore_axis_name="s"),
    scratch_types=[
        pltpu.VMEM((EPT,), jnp.int32),
        pltpu.VMEM((CHUNK,), jnp.float32),
        pltpu.VMEM((SLICE,), jnp.float32),
        pltpu.VMEM_SHARED((NPAD,), jnp.float32),
        pltpu.SemaphoreType.DMA,
        pltpu.SemaphoreType.DMA,
    ],
)


def kernel(x, We_in, We_out, w_ih, w_hh, b_ih, b_hh, edge_index_in,
           edge_index_out):
    deg = _sc_masks(edge_index_in[1].astype(jnp.int32),
                    edge_index_out[1].astype(jnp.int32))
    return _dense_stage(x, deg, We_in, We_out, w_ih, w_hh,
                        b_ih.reshape(1, 3 * D), b_hh.reshape(1, 3 * D))


# speedup vs baseline: 1.3970x; 1.1028x over previous
import functools

import jax
import jax.numpy as jnp
from jax import lax
from jax.experimental import pallas as pl
from jax.experimental.pallas import tpu as pltpu
from jax.experimental.pallas import tpu_sc as plsc

N_NODES = 10000
D = 128
TIMESTEP = 3

ROW_BLK = 2000

NTILES = 16
CHUNK = 128
EDGES = 160000
NCHUNK = 80
EPT = NCHUNK * CHUNK
STRIDE = 9984
NPAD = 10240
SLICE = NPAD // NTILES


def _tc_body(x_ref, deg_ref, win_ref, wout_ref, wih_ref, whh_ref,
             bih_ref, bhh_ref, out_ref, degt_scr):
    i = pl.program_id(0)
    xb = x_ref[...]
    dn = (((1,), (1,)), ((), ()))

    @pl.when(i == 0)
    def _():
        degt_scr[...] = jnp.transpose(deg_ref[...])

    degb = degt_scr[pl.ds(i * ROW_BLK, ROW_BLK), :]
    bf = jnp.bfloat16
    xb16 = xb.astype(bf)
    pin = lax.dot_general(xb16, win_ref[...].astype(bf), dn,
                          preferred_element_type=jnp.float32)
    pout = lax.dot_general(xb16, wout_ref[...].astype(bf), dn,
                           preferred_element_type=jnp.float32)
    a_in = jnp.where(degb[:, 0:1] > 0.0, pin, 0.0)
    a_out = jnp.where(degb[:, 1:2] > 0.0, pout, 0.0)
    gf = jnp.maximum(a_in, a_out)
    wih16 = wih_ref[...].astype(bf)
    whh16 = whh_ref[...].astype(bf)
    gi = lax.dot_general(gf.astype(bf), wih16, dn,
                         preferred_element_type=jnp.float32) + bih_ref[...]
    h = xb
    for _ in range(TIMESTEP):
        gh = lax.dot_general(h.astype(bf), whh16, dn,
                             preferred_element_type=jnp.float32) + bhh_ref[...]
        r = jax.nn.sigmoid(gi[:, :D] + gh[:, :D])
        z = jax.nn.sigmoid(gi[:, D:2 * D] + gh[:, D:2 * D])
        n = jnp.tanh(gi[:, 2 * D:] + r * gh[:, 2 * D:])
        h = (1.0 - z) * n + z * h
    out_ref[...] = h


def _dense_stage(x, deg, We_in, We_out, w_ih, w_hh, b_ih, b_hh):
    n = x.shape[0]
    grid = n // ROW_BLK
    wspec = lambda a: pl.BlockSpec(a.shape, lambda i: (0,) * a.ndim)
    return pl.pallas_call(
        _tc_body,
        grid=(grid,),
        in_specs=[
            pl.BlockSpec((ROW_BLK, D), lambda i: (i, 0)),
            wspec(deg),
            wspec(We_in), wspec(We_out), wspec(w_ih), wspec(w_hh),
            wspec(b_ih), wspec(b_hh),
        ],
        out_specs=pl.BlockSpec((ROW_BLK, D), lambda i: (i, 0)),
        out_shape=jax.ShapeDtypeStruct((n, D), jnp.float32),
        scratch_shapes=[pltpu.VMEM((NPAD, 2), jnp.float32)],
    )(x, deg, We_in, We_out, w_ih, w_hh, b_ih, b_hh)


def _sc_mask_body(ein_ref, eout_ref, out_ref, idx_v, ones_v, zeros_v,
                  shared, idx_sem, sc_sem):
    c = lax.axis_index("c")
    s = lax.axis_index("s")
    base = pl.multiple_of(s * STRIDE, CHUNK)

    def start_stage(e_ref):
        def body(j, carry):
            pltpu.make_async_copy(
                e_ref.at[:, pl.ds(base + j * CHUNK, CHUNK)],
                idx_v.at[j], idx_sem).start()
            return carry
        lax.fori_loop(0, NCHUNK, body, 0)

    @pl.when(c == 0)
    def _():
        start_stage(ein_ref)

    @pl.when(c == 1)
    def _():
        start_stage(eout_ref)

    for i in range(CHUNK // 16):
        ones_v[pl.ds(i * 16, 16)] = jnp.full((16,), 1.0, jnp.float32)
    for i in range(SLICE // 16):
        zeros_v[pl.ds(i * 16, 16)] = jnp.zeros((16,), jnp.float32)
    pltpu.sync_copy(zeros_v, shared.at[pl.ds(s * SLICE, SLICE)])

    def wait_stage(j, carry):
        pltpu.make_async_copy(ein_ref.at[:, pl.ds(base, CHUNK)],
                              idx_v.at[0], idx_sem).wait()
        return carry

    lax.fori_loop(0, NCHUNK, wait_stage, 0)
    plsc.subcore_barrier()

    def fire(j, carry):
        pltpu.async_copy(ones_v, shared.at[idx_v.at[j, 1]], sc_sem, add=True)
        return carry

    lax.fori_loop(0, NCHUNK, fire, 0)

    def drain(j, carry):
        pltpu.make_async_copy(ones_v, shared.at[idx_v.at[0, 1]],
                              sc_sem).wait()
        return carry

    lax.fori_loop(0, NCHUNK, drain, 0)

    plsc.subcore_barrier()
    pltpu.sync_copy(shared.at[pl.ds(s * SLICE, SLICE)],
                    out_ref.at[c, pl.ds(s * SLICE, SLICE)])


_sc_masks = pl.kernel(
    _sc_mask_body,
    out_type=jax.ShapeDtypeStruct((2, NPAD), jnp.float32),
    mesh=plsc.VectorSubcoreMesh(core_axis_name="c", subcore_axis_name="s"),
    scratch_types=[
        pltpu.VMEM((NCHUNK, 2, CHUNK), jnp.int32),
        pltpu.VMEM((CHUNK,), jnp.float32),
        pltpu.VMEM((SLICE,), jnp.float32),
        pltpu.VMEM_SHARED((NPAD,), jnp.float32),
        pltpu.SemaphoreType.DMA,
        pltpu.SemaphoreType.DMA,
    ],
)


def kernel(x, We_in, We_out, w_ih, w_hh, b_ih, b_hh, edge_index_in,
           edge_index_out):
    deg = _sc_masks(edge_index_in.astype(jnp.int32),
                    edge_index_out.astype(jnp.int32))
    return _dense_stage(x, deg, We_in, We_out, w_ih, w_hh,
                        b_ih.reshape(1, 3 * D), b_hh.reshape(1, 3 * D))


# speedup vs baseline: 1.4239x; 1.0193x over previous
import functools

import jax
import jax.numpy as jnp
from jax import lax
from jax.experimental import pallas as pl
from jax.experimental.pallas import tpu as pltpu
from jax.experimental.pallas import tpu_sc as plsc

N_NODES = 10000
D = 128
TIMESTEP = 3

ROW_BLK = 2000

NTILES = 16
CHUNK = 128
EDGES = 160000
NCHUNK = 80
EPT = NCHUNK * CHUNK
STRIDE = 9984
NPAD = 10240
SLICE = NPAD // NTILES


def _tc_body(x_ref, deg_ref, win_ref, wout_ref, wih_ref, whh_ref,
             bih_ref, bhh_ref, out_ref, degt_scr):
    i = pl.program_id(0)
    xb = x_ref[...]
    dn = (((1,), (1,)), ((), ()))

    @pl.when(i == 0)
    def _():
        degt_scr[...] = jnp.transpose(deg_ref[...])

    degb = degt_scr[pl.ds(i * ROW_BLK, ROW_BLK), :]
    bf = jnp.bfloat16
    xb16 = xb.astype(bf)
    wio16 = jnp.concatenate([win_ref[...], wout_ref[...]], axis=0).astype(bf)
    po = lax.dot_general(xb16, wio16, dn,
                         preferred_element_type=jnp.float32)
    a_in = jnp.where(degb[:, 0:1] > 0.0, po[:, :D], 0.0)
    a_out = jnp.where(degb[:, 1:2] > 0.0, po[:, D:], 0.0)
    gf = jnp.maximum(a_in, a_out)
    wih16 = wih_ref[...].astype(bf)
    whh16 = whh_ref[...].astype(bf)
    bih16 = bih_ref[...].astype(bf)
    bhh16 = bhh_ref[...].astype(bf)
    gi = lax.dot_general(gf.astype(bf), wih16, dn,
                         preferred_element_type=jnp.float32).astype(bf) + bih16
    h = xb16
    for _ in range(TIMESTEP):
        gh = lax.dot_general(h, whh16, dn,
                             preferred_element_type=jnp.float32).astype(bf) + bhh16
        r = jax.nn.sigmoid(gi[:, :D] + gh[:, :D])
        z = jax.nn.sigmoid(gi[:, D:2 * D] + gh[:, D:2 * D])
        n = jnp.tanh(gi[:, 2 * D:] + r * gh[:, 2 * D:])
        h = n + z * (h - n)
    out_ref[...] = h.astype(jnp.float32)


def _dense_stage(x, deg, We_in, We_out, w_ih, w_hh, b_ih, b_hh):
    n = x.shape[0]
    grid = n // ROW_BLK
    wspec = lambda a: pl.BlockSpec(a.shape, lambda i: (0,) * a.ndim)
    return pl.pallas_call(
        _tc_body,
        grid=(grid,),
        in_specs=[
            pl.BlockSpec((ROW_BLK, D), lambda i: (i, 0)),
            wspec(deg),
            wspec(We_in), wspec(We_out), wspec(w_ih), wspec(w_hh),
            wspec(b_ih), wspec(b_hh),
        ],
        out_specs=pl.BlockSpec((ROW_BLK, D), lambda i: (i, 0)),
        out_shape=jax.ShapeDtypeStruct((n, D), jnp.float32),
        scratch_shapes=[pltpu.VMEM((NPAD, 2), jnp.float32)],
    )(x, deg, We_in, We_out, w_ih, w_hh, b_ih, b_hh)


def _sc_mask_body(ein_ref, eout_ref, out_ref, idx_v, ones_v, zeros_v,
                  shared, idx_sem, sc_sem):
    c = lax.axis_index("c")
    s = lax.axis_index("s")
    base = pl.multiple_of(s * STRIDE, CHUNK)

    def start_stage(e_ref):
        def body(j, carry):
            pltpu.make_async_copy(
                e_ref.at[:, pl.ds(base + j * CHUNK, CHUNK)],
                idx_v.at[j], idx_sem).start()
            return carry
        lax.fori_loop(0, NCHUNK, body, 0)

    @pl.when(c == 0)
    def _():
        start_stage(ein_ref)

    @pl.when(c == 1)
    def _():
        start_stage(eout_ref)

    for i in range(CHUNK // 16):
        ones_v[pl.ds(i * 16, 16)] = jnp.full((16,), 1.0, jnp.float32)
    for i in range(SLICE // 16):
        zeros_v[pl.ds(i * 16, 16)] = jnp.zeros((16,), jnp.float32)
    pltpu.sync_copy(zeros_v, shared.at[pl.ds(s * SLICE, SLICE)])

    def wait_stage(j, carry):
        pltpu.make_async_copy(ein_ref.at[:, pl.ds(base, CHUNK)],
                              idx_v.at[0], idx_sem).wait()
        return carry

    lax.fori_loop(0, NCHUNK, wait_stage, 0)
    plsc.subcore_barrier()

    def fire(j, carry):
        pltpu.async_copy(ones_v, shared.at[idx_v.at[j, 1]], sc_sem, add=True)
        return carry

    lax.fori_loop(0, NCHUNK, fire, 0)

    def drain(j, carry):
        pltpu.make_async_copy(ones_v, shared.at[idx_v.at[0, 1]],
                              sc_sem).wait()
        return carry

    lax.fori_loop(0, NCHUNK, drain, 0)

    plsc.subcore_barrier()
    pltpu.sync_copy(shared.at[pl.ds(s * SLICE, SLICE)],
                    out_ref.at[c, pl.ds(s * SLICE, SLICE)])


_sc_masks = pl.kernel(
    _sc_mask_body,
    out_type=jax.ShapeDtypeStruct((2, NPAD), jnp.float32),
    mesh=plsc.VectorSubcoreMesh(core_axis_name="c", subcore_axis_name="s"),
    scratch_types=[
        pltpu.VMEM((NCHUNK, 2, CHUNK), jnp.int32),
        pltpu.VMEM((CHUNK,), jnp.float32),
        pltpu.VMEM((SLICE,), jnp.float32),
        pltpu.VMEM_SHARED((NPAD,), jnp.float32),
        pltpu.SemaphoreType.DMA,
        pltpu.SemaphoreType.DMA,
    ],
)


def kernel(x, We_in, We_out, w_ih, w_hh, b_ih, b_hh, edge_index_in,
           edge_index_out):
    deg = _sc_masks(edge_index_in.astype(jnp.int32),
                    edge_index_out.astype(jnp.int32))
    return _dense_stage(x, deg, We_in, We_out, w_ih, w_hh,
                        b_ih.reshape(1, 3 * D), b_hh.reshape(1, 3 * D))


# speedup vs baseline: 1.4296x; 1.0040x over previous
import functools

import jax
import jax.numpy as jnp
from jax import lax
from jax.experimental import pallas as pl
from jax.experimental.pallas import tpu as pltpu
from jax.experimental.pallas import tpu_sc as plsc

N_NODES = 10000
D = 128
TIMESTEP = 3

ROW_BLK = 2000

NTILES = 16
CHUNK = 128
EDGES = 160000
NCHUNK = 80
EPT = NCHUNK * CHUNK
STRIDE = 9984
NPAD = 10240
SLICE = NPAD // NTILES


def _tc_body(x_ref, deg_ref, win_ref, wout_ref, wih_ref, whh_ref,
             bih_ref, bhh_ref, out_ref, degt_scr):
    i = pl.program_id(0)
    xb = x_ref[...]
    dn = (((1,), (1,)), ((), ()))

    @pl.when(i == 0)
    def _():
        degt_scr[...] = jnp.transpose(deg_ref[...])

    degb = degt_scr[pl.ds(i * ROW_BLK, ROW_BLK), :]
    bf = jnp.bfloat16
    xb16 = xb.astype(bf)
    wio16 = jnp.concatenate([win_ref[...], wout_ref[...]], axis=0).astype(bf)
    po = lax.dot_general(xb16, wio16, dn,
                         preferred_element_type=jnp.float32)
    a_in = jnp.where(degb[:, 0:1] > 0.0, po[:, :D], 0.0)
    a_out = jnp.where(degb[:, 1:2] > 0.0, po[:, D:], 0.0)
    gf = jnp.maximum(a_in, a_out)
    wih16 = wih_ref[...].astype(bf)
    whh16 = whh_ref[...].astype(bf)

    def sigmoid(v):
        return 0.5 * jnp.tanh(0.5 * v) + 0.5

    gi = lax.dot_general(gf.astype(bf), wih16, dn,
                         preferred_element_type=jnp.float32) + bih_ref[...]
    h = xb
    for _ in range(TIMESTEP):
        gh = lax.dot_general(h.astype(bf), whh16, dn,
                             preferred_element_type=jnp.float32) + bhh_ref[...]
        r = sigmoid(gi[:, :D] + gh[:, :D])
        z = sigmoid(gi[:, D:2 * D] + gh[:, D:2 * D])
        n = jnp.tanh(gi[:, 2 * D:] + r * gh[:, 2 * D:])
        h = n + z * (h - n)
    out_ref[...] = h


def _dense_stage(x, deg, We_in, We_out, w_ih, w_hh, b_ih, b_hh):
    n = x.shape[0]
    grid = n // ROW_BLK
    wspec = lambda a: pl.BlockSpec(a.shape, lambda i: (0,) * a.ndim)
    return pl.pallas_call(
        _tc_body,
        grid=(grid,),
        in_specs=[
            pl.BlockSpec((ROW_BLK, D), lambda i: (i, 0)),
            wspec(deg),
            wspec(We_in), wspec(We_out), wspec(w_ih), wspec(w_hh),
            wspec(b_ih), wspec(b_hh),
        ],
        out_specs=pl.BlockSpec((ROW_BLK, D), lambda i: (i, 0)),
        out_shape=jax.ShapeDtypeStruct((n, D), jnp.float32),
        scratch_shapes=[pltpu.VMEM((NPAD, 2), jnp.float32)],
    )(x, deg, We_in, We_out, w_ih, w_hh, b_ih, b_hh)


def _sc_mask_body(ein_ref, eout_ref, out_ref, idx_v, ones_v, zeros_v,
                  shared, idx_sem, sc_sem):
    c = lax.axis_index("c")
    s = lax.axis_index("s")
    base = pl.multiple_of(s * STRIDE, CHUNK)

    def start_stage(e_ref):
        def body(j, carry):
            pltpu.make_async_copy(
                e_ref.at[:, pl.ds(base + j * CHUNK, CHUNK)],
                idx_v.at[j], idx_sem).start()
            return carry
        lax.fori_loop(0, NCHUNK, body, 0)

    @pl.when(c == 0)
    def _():
        start_stage(ein_ref)

    @pl.when(c == 1)
    def _():
        start_stage(eout_ref)

    for i in range(CHUNK // 16):
        ones_v[pl.ds(i * 16, 16)] = jnp.full((16,), 1.0, jnp.float32)
    for i in range(SLICE // 16):
        zeros_v[pl.ds(i * 16, 16)] = jnp.zeros((16,), jnp.float32)
    pltpu.sync_copy(zeros_v, shared.at[pl.ds(s * SLICE, SLICE)])

    def wait_stage(j, carry):
        pltpu.make_async_copy(ein_ref.at[:, pl.ds(base, CHUNK)],
                              idx_v.at[0], idx_sem).wait()
        return carry

    lax.fori_loop(0, NCHUNK, wait_stage, 0)
    plsc.subcore_barrier()

    def fire(j, carry):
        pltpu.async_copy(ones_v, shared.at[idx_v.at[j, 1]], sc_sem, add=True)
        return carry

    lax.fori_loop(0, NCHUNK, fire, 0)

    def drain(j, carry):
        pltpu.make_async_copy(ones_v, shared.at[idx_v.at[0, 1]],
                              sc_sem).wait()
        return carry

    lax.fori_loop(0, NCHUNK, drain, 0)

    plsc.subcore_barrier()
    pltpu.sync_copy(shared.at[pl.ds(s * SLICE, SLICE)],
                    out_ref.at[c, pl.ds(s * SLICE, SLICE)])


_sc_masks = pl.kernel(
    _sc_mask_body,
    out_type=jax.ShapeDtypeStruct((2, NPAD), jnp.float32),
    mesh=plsc.VectorSubcoreMesh(core_axis_name="c", subcore_axis_name="s"),
    scratch_types=[
        pltpu.VMEM((NCHUNK, 2, CHUNK), jnp.int32),
        pltpu.VMEM((CHUNK,), jnp.float32),
        pltpu.VMEM((SLICE,), jnp.float32),
        pltpu.VMEM_SHARED((NPAD,), jnp.float32),
        pltpu.SemaphoreType.DMA,
        pltpu.SemaphoreType.DMA,
    ],
)


def kernel(x, We_in, We_out, w_ih, w_hh, b_ih, b_hh, edge_index_in,
           edge_index_out):
    deg = _sc_masks(edge_index_in.astype(jnp.int32),
                    edge_index_out.astype(jnp.int32))
    return _dense_stage(x, deg, We_in, We_out, w_ih, w_hh,
                        b_ih.reshape(1, 3 * D), b_hh.reshape(1, 3 * D))
